# baseline (speedup 1.0000x reference)
import functools

import jax
import jax.numpy as jnp
from jax import lax
from jax.experimental import pallas as pl
from jax.experimental.pallas import tpu as pltpu
from jax.experimental.pallas import tpu_sc as plsc

E = 8
K = 2
B = 2
S = 2048
D = 1024
F = 4096
T = B * S

M = 256
FB = 2048
NF = F // FB
NT = 40
NPAD = NT * M


def _gate_body(x_ref, wg_ref, bg_ref, w_ref, idx_ref):
    x = x_ref[0]
    logits = jax.lax.dot_general(
        x, wg_ref[...], (((1,), (0,)), ((), ())),
        preferred_element_type=jnp.float32) + bg_ref[...]
    iota = jax.lax.broadcasted_iota(jnp.int32, logits.shape, 1)
    m1 = jnp.max(logits, axis=1, keepdims=True)
    a1 = jnp.min(jnp.where(logits == m1, iota, E), axis=1, keepdims=True)
    l2 = jnp.where(iota == a1, -jnp.inf, logits)
    m2 = jnp.max(l2, axis=1, keepdims=True)
    a2 = jnp.min(jnp.where(l2 == m2, iota, E), axis=1, keepdims=True)
    e1 = jnp.exp(m1 - jnp.max(m1, axis=0, keepdims=True))
    e2 = jnp.exp(m2 - jnp.max(m2, axis=0, keepdims=True))
    w_ref[0, :, 0:1] = e1 / jnp.sum(e1, axis=0, keepdims=True)
    w_ref[0, :, 1:2] = e2 / jnp.sum(e2, axis=0, keepdims=True)
    idx_ref[0, :, 0:1] = a1
    idx_ref[0, :, 1:2] = a2


def _route(inputs, Wg, bg):
    return pl.pallas_call(
        _gate_body,
        grid=(B,),
        in_specs=[
            pl.BlockSpec((1, S, D), lambda b: (b, 0, 0)),
            pl.BlockSpec((D, E), lambda b: (0, 0)),
            pl.BlockSpec((1, E), lambda b: (0, 0)),
        ],
        out_specs=[
            pl.BlockSpec((1, S, K), lambda b: (b, 0, 0)),
            pl.BlockSpec((1, S, K), lambda b: (b, 0, 0)),
        ],
        out_shape=[
            jax.ShapeDtypeStruct((B, S, K), jnp.float32),
            jax.ShapeDtypeStruct((B, S, K), jnp.int32),
        ],
    )(inputs, Wg, bg.reshape(1, E))


def _ffn_body(meta_ref, xs_ref, w1_ref, b1_ref, w2_ref, b2_ref, ws_ref, y_ref):
    t = pl.program_id(0)
    f = pl.program_id(1)

    @pl.when(meta_ref[NT + t] == 1)
    def _():
        x = xs_ref[...].astype(jnp.bfloat16)
        h = jax.lax.dot_general(
            x, w1_ref[0], (((1,), (0,)), ((), ())),
            preferred_element_type=jnp.float32) + b1_ref[0]
        h = 0.5 * h * (1.0 + jax.lax.erf(h * 0.7071067811865476))
        part = jax.lax.dot_general(
            h.astype(jnp.bfloat16), w2_ref[0], (((1,), (0,)), ((), ())),
            preferred_element_type=jnp.float32)

        @pl.when(f == 0)
        def _():
            y_ref[...] = part + b2_ref[0]

        @pl.when(f > 0)
        def _():
            y_ref[...] = y_ref[...] + part

        @pl.when(f == NF - 1)
        def _():
            y_ref[...] = y_ref[...] * ws_ref[...]


def _grouped_ffn(meta, xs, W1, b1, W2, b2, ws):
    grid_spec = pltpu.PrefetchScalarGridSpec(
        num_scalar_prefetch=1,
        grid=(NT, NF),
        in_specs=[
            pl.BlockSpec((M, D), lambda t, f, m: (t, 0)),
            pl.BlockSpec((1, D, FB), lambda t, f, m: (m[t], 0, f)),
            pl.BlockSpec((1, 1, FB), lambda t, f, m: (m[t], 0, f)),
            pl.BlockSpec((1, FB, D), lambda t, f, m: (m[t], f, 0)),
            pl.BlockSpec((1, 1, D), lambda t, f, m: (m[t], 0, 0)),
            pl.BlockSpec((M, 1), lambda t, f, m: (t, 0)),
        ],
        out_specs=pl.BlockSpec((M, D), lambda t, f, m: (t, 0)),
    )
    return pl.pallas_call(
        _ffn_body,
        grid_spec=grid_spec,
        out_shape=jax.ShapeDtypeStruct((NPAD, D), jnp.float32),
        compiler_params=pltpu.CompilerParams(
            dimension_semantics=("arbitrary", "arbitrary")),
    )(meta, xs, W1.astype(jnp.bfloat16), b1.reshape(E, 1, F),
      W2.astype(jnp.bfloat16), b2.reshape(E, 1, D), ws)


NW = 32
GCH = 64
CCH = 32

_sc_mesh = plsc.VectorSubcoreMesh(core_axis_name="c", subcore_axis_name="s")


@functools.partial(
    pl.kernel, mesh=_sc_mesh,
    out_type=jax.ShapeDtypeStruct((NPAD, D), jnp.float32),
    scratch_types=[
        pltpu.VMEM((GCH,), jnp.int32),
        pltpu.VMEM((GCH, D), jnp.float32),
        pltpu.SemaphoreType.DMA,
    ],
)
def _sc_dispatch(x_hbm, tok_hbm, xs_hbm, idx_v, rows_v, sem):
    wid = lax.axis_index("s") * 2 + lax.axis_index("c")
    base = wid * (NPAD // NW)
    for c in range(NPAD // NW // GCH):
        off = base + c * GCH
        pltpu.sync_copy(tok_hbm.at[pl.ds(off, GCH)], idx_v)
        pltpu.async_copy(x_hbm.at[idx_v], rows_v, sem).wait()
        pltpu.sync_copy(rows_v, xs_hbm.at[pl.ds(off, GCH)])


@functools.partial(
    pl.kernel, mesh=_sc_mesh,
    out_type=jax.ShapeDtypeStruct((T, D), jnp.float32),
    scratch_types=[
        pltpu.VMEM((CCH,), jnp.int32),
        pltpu.VMEM((CCH,), jnp.int32),
        pltpu.VMEM((CCH, D), jnp.float32),
        pltpu.VMEM((CCH, D), jnp.float32),
        pltpu.SemaphoreType.DMA,
        pltpu.SemaphoreType.DMA,
    ],
)
def _sc_combine(y_hbm, p0_hbm, p1_hbm, out_hbm, i0_v, i1_v, y0_v, y1_v, s0, s1):
    wid = lax.axis_index("s") * 2 + lax.axis_index("c")
    base = wid * (T // NW)
    for c in range(T // NW // CCH):
        off = base + c * CCH
        pltpu.sync_copy(p0_hbm.at[pl.ds(off, CCH)], i0_v)
        pltpu.sync_copy(p1_hbm.at[pl.ds(off, CCH)], i1_v)
        cp0 = pltpu.async_copy(y_hbm.at[i0_v], y0_v, s0)
        cp1 = pltpu.async_copy(y_hbm.at[i1_v], y1_v, s1)
        cp0.wait()
        cp1.wait()

        def _row(r, carry):
            for j in range(D // 16):
                sl = pl.ds(j * 16, 16)
                y0_v[r, sl] = y0_v[r, sl] + y1_v[r, sl]
            return carry

        lax.fori_loop(0, CCH, _row, 0)
        pltpu.sync_copy(y0_v, out_hbm.at[pl.ds(off, CCH)])


def kernel(inputs, Wg, bg, W1, b1, W2, b2):
    x_flat = inputs.reshape(T, D)

    w_bsk, idx_bsk = _route(inputs, Wg, bg)
    wf = w_bsk.reshape(T * K)
    e_flat = idx_bsk.reshape(T * K)

    oh = (e_flat[:, None] == jnp.arange(E, dtype=jnp.int32)).astype(jnp.int32)
    ranks = jnp.cumsum(oh, axis=0)
    counts = ranks[-1]
    rank = jnp.take_along_axis(ranks, e_flat[:, None], axis=1)[:, 0] - 1
    tiles_per_e = (counts + M - 1) // M
    seg_starts = jnp.concatenate(
        [jnp.zeros(1, jnp.int32),
         jnp.cumsum(tiles_per_e).astype(jnp.int32)]) * M
    pos = seg_starts[e_flat] + rank
    arange_a = jnp.arange(T * K, dtype=jnp.int32)
    row_tok = jnp.zeros(NPAD, jnp.int32).at[pos].set(arange_a // K)
    ws = jnp.zeros(NPAD, jnp.float32).at[pos].set(wf)
    num_real = jnp.sum(tiles_per_e).astype(jnp.int32)
    tile_ids = jnp.arange(NT, dtype=jnp.int32)
    tile_eid = jnp.minimum(
        jnp.searchsorted(seg_starts[1:], tile_ids * M, side="right"),
        E - 1).astype(jnp.int32)
    tile_valid = (tile_ids < num_real).astype(jnp.int32)
    meta = jnp.concatenate([tile_eid, tile_valid])

    xs = _sc_dispatch(x_flat, row_tok)

    y = _grouped_ffn(meta, xs, W1, b1, W2, b2, ws.reshape(NPAD, 1))

    pos_tk = pos.reshape(T, K)
    out = _sc_combine(y, pos_tk[:, 0], pos_tk[:, 1])
    return out.reshape(B, S, D)


# speedup vs baseline: 1.0337x; 1.0337x over previous
import functools

import jax
import jax.numpy as jnp
from jax import lax
from jax.experimental import pallas as pl
from jax.experimental.pallas import tpu as pltpu
from jax.experimental.pallas import tpu_sc as plsc

E = 8
K = 2
B = 2
S = 2048
D = 1024
F = 4096
T = B * S

M = 256
FB = 2048
NF = F // FB
NT = 40
NPAD = NT * M


def _gate_body(x_ref, wg_ref, bg_ref, w_ref, idx_ref):
    x = x_ref[0]
    logits = jax.lax.dot_general(
        x, wg_ref[...], (((1,), (0,)), ((), ())),
        preferred_element_type=jnp.float32) + bg_ref[...]
    iota = jax.lax.broadcasted_iota(jnp.int32, logits.shape, 1)
    m1 = jnp.max(logits, axis=1, keepdims=True)
    a1 = jnp.min(jnp.where(logits == m1, iota, E), axis=1, keepdims=True)
    l2 = jnp.where(iota == a1, -jnp.inf, logits)
    m2 = jnp.max(l2, axis=1, keepdims=True)
    a2 = jnp.min(jnp.where(l2 == m2, iota, E), axis=1, keepdims=True)
    e1 = jnp.exp(m1 - jnp.max(m1, axis=0, keepdims=True))
    e2 = jnp.exp(m2 - jnp.max(m2, axis=0, keepdims=True))
    w_ref[0, :, 0:1] = e1 / jnp.sum(e1, axis=0, keepdims=True)
    w_ref[0, :, 1:2] = e2 / jnp.sum(e2, axis=0, keepdims=True)
    idx_ref[0, :, 0:1] = a1
    idx_ref[0, :, 1:2] = a2


def _route(inputs, Wg, bg):
    return pl.pallas_call(
        _gate_body,
        grid=(B,),
        in_specs=[
            pl.BlockSpec((1, S, D), lambda b: (b, 0, 0)),
            pl.BlockSpec((D, E), lambda b: (0, 0)),
            pl.BlockSpec((1, E), lambda b: (0, 0)),
        ],
        out_specs=[
            pl.BlockSpec((1, S, K), lambda b: (b, 0, 0)),
            pl.BlockSpec((1, S, K), lambda b: (b, 0, 0)),
        ],
        out_shape=[
            jax.ShapeDtypeStruct((B, S, K), jnp.float32),
            jax.ShapeDtypeStruct((B, S, K), jnp.int32),
        ],
    )(inputs, Wg, bg.reshape(1, E))


def _ffn_body(meta_ref, xs_ref, w1_ref, b1_ref, w2_ref, b2_ref, ws_ref, y_ref):
    t = pl.program_id(0)
    f = pl.program_id(1)

    @pl.when(meta_ref[NT + t] == 1)
    def _():
        x = xs_ref[...]
        h = jax.lax.dot_general(
            x, w1_ref[0], (((1,), (0,)), ((), ())),
            preferred_element_type=jnp.float32) + b1_ref[0]
        h = 0.5 * h * (1.0 + jax.lax.erf(h * 0.7071067811865476))
        part = jax.lax.dot_general(
            h, w2_ref[0], (((1,), (0,)), ((), ())),
            preferred_element_type=jnp.float32)

        @pl.when(f == 0)
        def _():
            y_ref[...] = part + b2_ref[0]

        @pl.when(f > 0)
        def _():
            y_ref[...] = y_ref[...] + part

        @pl.when(f == NF - 1)
        def _():
            y_ref[...] = y_ref[...] * ws_ref[...]


def _grouped_ffn(meta, xs, W1, b1, W2, b2, ws):
    grid_spec = pltpu.PrefetchScalarGridSpec(
        num_scalar_prefetch=1,
        grid=(NT, NF),
        in_specs=[
            pl.BlockSpec((M, D), lambda t, f, m: (t, 0)),
            pl.BlockSpec((1, D, FB), lambda t, f, m: (m[t], 0, f)),
            pl.BlockSpec((1, 1, FB), lambda t, f, m: (m[t], 0, f)),
            pl.BlockSpec((1, FB, D), lambda t, f, m: (m[t], f, 0)),
            pl.BlockSpec((1, 1, D), lambda t, f, m: (m[t], 0, 0)),
            pl.BlockSpec((M, 1), lambda t, f, m: (t, 0)),
        ],
        out_specs=pl.BlockSpec((M, D), lambda t, f, m: (t, 0)),
    )
    return pl.pallas_call(
        _ffn_body,
        grid_spec=grid_spec,
        out_shape=jax.ShapeDtypeStruct((NPAD, D), jnp.float32),
        compiler_params=pltpu.CompilerParams(
            dimension_semantics=("arbitrary", "arbitrary")),
    )(meta, xs, W1, b1.reshape(E, 1, F), W2, b2.reshape(E, 1, D), ws)


NW = 32
GCH = 64
CCH = 32

_sc_mesh = plsc.VectorSubcoreMesh(core_axis_name="c", subcore_axis_name="s")


@functools.partial(
    pl.kernel, mesh=_sc_mesh,
    out_type=jax.ShapeDtypeStruct((NPAD, D), jnp.float32),
    scratch_types=[
        pltpu.VMEM((GCH,), jnp.int32),
        pltpu.VMEM((GCH, D), jnp.float32),
        pltpu.SemaphoreType.DMA,
    ],
)
def _sc_dispatch(x_hbm, tok_hbm, xs_hbm, idx_v, rows_v, sem):
    wid = lax.axis_index("s") * 2 + lax.axis_index("c")
    base = wid * (NPAD // NW)
    for c in range(NPAD // NW // GCH):
        off = base + c * GCH
        pltpu.sync_copy(tok_hbm.at[pl.ds(off, GCH)], idx_v)
        pltpu.async_copy(x_hbm.at[idx_v], rows_v, sem).wait()
        pltpu.sync_copy(rows_v, xs_hbm.at[pl.ds(off, GCH)])


@functools.partial(
    pl.kernel, mesh=_sc_mesh,
    out_type=jax.ShapeDtypeStruct((T, D), jnp.float32),
    scratch_types=[
        pltpu.VMEM((CCH,), jnp.int32),
        pltpu.VMEM((CCH,), jnp.int32),
        pltpu.VMEM((CCH, D), jnp.float32),
        pltpu.VMEM((CCH, D), jnp.float32),
        pltpu.SemaphoreType.DMA,
        pltpu.SemaphoreType.DMA,
    ],
)
def _sc_combine(y_hbm, p0_hbm, p1_hbm, out_hbm, i0_v, i1_v, y0_v, y1_v, s0, s1):
    wid = lax.axis_index("s") * 2 + lax.axis_index("c")
    base = wid * (T // NW)
    for c in range(T // NW // CCH):
        off = base + c * CCH
        pltpu.sync_copy(p0_hbm.at[pl.ds(off, CCH)], i0_v)
        pltpu.sync_copy(p1_hbm.at[pl.ds(off, CCH)], i1_v)
        cp0 = pltpu.async_copy(y_hbm.at[i0_v], y0_v, s0)
        cp1 = pltpu.async_copy(y_hbm.at[i1_v], y1_v, s1)
        cp0.wait()
        cp1.wait()

        def _row(r, carry):
            for j in range(D // 16):
                sl = pl.ds(j * 16, 16)
                y0_v[r, sl] = y0_v[r, sl] + y1_v[r, sl]
            return carry

        lax.fori_loop(0, CCH, _row, 0)
        pltpu.sync_copy(y0_v, out_hbm.at[pl.ds(off, CCH)])


def kernel(inputs, Wg, bg, W1, b1, W2, b2):
    x_flat = inputs.reshape(T, D)

    w_bsk, idx_bsk = _route(inputs, Wg, bg)
    wf = w_bsk.reshape(T * K)
    e_flat = idx_bsk.reshape(T * K)

    oh = (e_flat[:, None] == jnp.arange(E, dtype=jnp.int32)).astype(jnp.int32)
    ranks = jnp.cumsum(oh, axis=0)
    counts = ranks[-1]
    rank = jnp.take_along_axis(ranks, e_flat[:, None], axis=1)[:, 0] - 1
    tiles_per_e = (counts + M - 1) // M
    seg_starts = jnp.concatenate(
        [jnp.zeros(1, jnp.int32),
         jnp.cumsum(tiles_per_e).astype(jnp.int32)]) * M
    pos = seg_starts[e_flat] + rank
    arange_a = jnp.arange(T * K, dtype=jnp.int32)
    row_tok = jnp.zeros(NPAD, jnp.int32).at[pos].set(arange_a // K)
    ws = jnp.zeros(NPAD, jnp.float32).at[pos].set(wf)
    num_real = jnp.sum(tiles_per_e).astype(jnp.int32)
    tile_ids = jnp.arange(NT, dtype=jnp.int32)
    tile_eid = jnp.minimum(
        jnp.searchsorted(seg_starts[1:], tile_ids * M, side="right"),
        E - 1).astype(jnp.int32)
    tile_valid = (tile_ids < num_real).astype(jnp.int32)
    meta = jnp.concatenate([tile_eid, tile_valid])

    xs = _sc_dispatch(x_flat, row_tok)

    y = _grouped_ffn(meta, xs, W1, b1, W2, b2, ws.reshape(NPAD, 1))

    pos_tk = pos.reshape(T, K)
    out = _sc_combine(y, pos_tk[:, 0], pos_tk[:, 1])
    return out.reshape(B, S, D)


# speedup vs baseline: 1.0461x; 1.0119x over previous
import functools

import jax
import jax.numpy as jnp
from jax import lax
from jax.experimental import pallas as pl
from jax.experimental.pallas import tpu as pltpu
from jax.experimental.pallas import tpu_sc as plsc

E = 8
K = 2
B = 2
S = 2048
D = 1024
F = 4096
T = B * S

M = 256
FB = 2048
NF = F // FB
NT = 40
NPAD = NT * M


def _gate_body(x_ref, wg_ref, bg_ref, w_ref, idx_ref):
    x = x_ref[0]
    logits = jax.lax.dot_general(
        x, wg_ref[...], (((1,), (0,)), ((), ())),
        preferred_element_type=jnp.float32) + bg_ref[...]
    iota = jax.lax.broadcasted_iota(jnp.int32, logits.shape, 1)
    m1 = jnp.max(logits, axis=1, keepdims=True)
    a1 = jnp.min(jnp.where(logits == m1, iota, E), axis=1, keepdims=True)
    l2 = jnp.where(iota == a1, -jnp.inf, logits)
    m2 = jnp.max(l2, axis=1, keepdims=True)
    a2 = jnp.min(jnp.where(l2 == m2, iota, E), axis=1, keepdims=True)
    e1 = jnp.exp(m1 - jnp.max(m1, axis=0, keepdims=True))
    e2 = jnp.exp(m2 - jnp.max(m2, axis=0, keepdims=True))
    w_ref[0, :, 0:1] = e1 / jnp.sum(e1, axis=0, keepdims=True)
    w_ref[0, :, 1:2] = e2 / jnp.sum(e2, axis=0, keepdims=True)
    idx_ref[0, :, 0:1] = a1
    idx_ref[0, :, 1:2] = a2


def _route(inputs, Wg, bg):
    return pl.pallas_call(
        _gate_body,
        grid=(B,),
        in_specs=[
            pl.BlockSpec((1, S, D), lambda b: (b, 0, 0)),
            pl.BlockSpec((D, E), lambda b: (0, 0)),
            pl.BlockSpec((1, E), lambda b: (0, 0)),
        ],
        out_specs=[
            pl.BlockSpec((1, S, K), lambda b: (b, 0, 0)),
            pl.BlockSpec((1, S, K), lambda b: (b, 0, 0)),
        ],
        out_shape=[
            jax.ShapeDtypeStruct((B, S, K), jnp.float32),
            jax.ShapeDtypeStruct((B, S, K), jnp.int32),
        ],
    )(inputs, Wg, bg.reshape(1, E))


def _ffn_body(meta_ref, xs_ref, w1_ref, b1_ref, w2_ref, b2_ref, ws_ref, y_ref):
    t = pl.program_id(0)
    f = pl.program_id(1)

    @pl.when(meta_ref[NT + t] == 1)
    def _():
        x = xs_ref[...].astype(jnp.bfloat16)
        h = jax.lax.dot_general(
            x, w1_ref[0].astype(jnp.bfloat16), (((1,), (0,)), ((), ())),
            preferred_element_type=jnp.float32) + b1_ref[0]
        h = 0.5 * h * (1.0 + jax.lax.erf(h * 0.7071067811865476))
        part = jax.lax.dot_general(
            h.astype(jnp.bfloat16), w2_ref[0].astype(jnp.bfloat16),
            (((1,), (0,)), ((), ())),
            preferred_element_type=jnp.float32)

        @pl.when(f == 0)
        def _():
            y_ref[...] = part + b2_ref[0]

        @pl.when(f > 0)
        def _():
            y_ref[...] = y_ref[...] + part

        @pl.when(f == NF - 1)
        def _():
            y_ref[...] = y_ref[...] * ws_ref[...]


def _grouped_ffn(meta, xs, W1, b1, W2, b2, ws):
    grid_spec = pltpu.PrefetchScalarGridSpec(
        num_scalar_prefetch=1,
        grid=(NT, NF),
        in_specs=[
            pl.BlockSpec((M, D), lambda t, f, m: (t, 0)),
            pl.BlockSpec((1, D, FB), lambda t, f, m: (m[t], 0, f)),
            pl.BlockSpec((1, 1, FB), lambda t, f, m: (m[t], 0, f)),
            pl.BlockSpec((1, FB, D), lambda t, f, m: (m[t], f, 0)),
            pl.BlockSpec((1, 1, D), lambda t, f, m: (m[t], 0, 0)),
            pl.BlockSpec((M, 1), lambda t, f, m: (t, 0)),
        ],
        out_specs=pl.BlockSpec((M, D), lambda t, f, m: (t, 0)),
    )
    return pl.pallas_call(
        _ffn_body,
        grid_spec=grid_spec,
        out_shape=jax.ShapeDtypeStruct((NPAD, D), jnp.float32),
        compiler_params=pltpu.CompilerParams(
            dimension_semantics=("arbitrary", "arbitrary")),
    )(meta, xs, W1, b1.reshape(E, 1, F), W2, b2.reshape(E, 1, D), ws)


NW = 32
GCH = 64
CCH = 32

_sc_mesh = plsc.VectorSubcoreMesh(core_axis_name="c", subcore_axis_name="s")


@functools.partial(
    pl.kernel, mesh=_sc_mesh,
    out_type=jax.ShapeDtypeStruct((NPAD, D), jnp.float32),
    scratch_types=[
        pltpu.VMEM((GCH,), jnp.int32),
        pltpu.VMEM((GCH, D), jnp.float32),
        pltpu.SemaphoreType.DMA,
    ],
)
def _sc_dispatch(x_hbm, tok_hbm, xs_hbm, idx_v, rows_v, sem):
    wid = lax.axis_index("s") * 2 + lax.axis_index("c")
    base = wid * (NPAD // NW)
    for c in range(NPAD // NW // GCH):
        off = base + c * GCH
        pltpu.sync_copy(tok_hbm.at[pl.ds(off, GCH)], idx_v)
        pltpu.async_copy(x_hbm.at[idx_v], rows_v, sem).wait()
        pltpu.sync_copy(rows_v, xs_hbm.at[pl.ds(off, GCH)])


@functools.partial(
    pl.kernel, mesh=_sc_mesh,
    out_type=jax.ShapeDtypeStruct((T, D), jnp.float32),
    scratch_types=[
        pltpu.VMEM((CCH,), jnp.int32),
        pltpu.VMEM((CCH,), jnp.int32),
        pltpu.VMEM((CCH, D), jnp.float32),
        pltpu.VMEM((CCH, D), jnp.float32),
        pltpu.SemaphoreType.DMA,
        pltpu.SemaphoreType.DMA,
    ],
)
def _sc_combine(y_hbm, p0_hbm, p1_hbm, out_hbm, i0_v, i1_v, y0_v, y1_v, s0, s1):
    wid = lax.axis_index("s") * 2 + lax.axis_index("c")
    base = wid * (T // NW)
    for c in range(T // NW // CCH):
        off = base + c * CCH
        pltpu.sync_copy(p0_hbm.at[pl.ds(off, CCH)], i0_v)
        pltpu.sync_copy(p1_hbm.at[pl.ds(off, CCH)], i1_v)
        cp0 = pltpu.async_copy(y_hbm.at[i0_v], y0_v, s0)
        cp1 = pltpu.async_copy(y_hbm.at[i1_v], y1_v, s1)
        cp0.wait()
        cp1.wait()

        def _row(r, carry):
            for j in range(D // 16):
                sl = pl.ds(j * 16, 16)
                y0_v[r, sl] = y0_v[r, sl] + y1_v[r, sl]
            return carry

        lax.fori_loop(0, CCH, _row, 0)
        pltpu.sync_copy(y0_v, out_hbm.at[pl.ds(off, CCH)])


def kernel(inputs, Wg, bg, W1, b1, W2, b2):
    x_flat = inputs.reshape(T, D)

    w_bsk, idx_bsk = _route(inputs, Wg, bg)
    wf = w_bsk.reshape(T * K)
    e_flat = idx_bsk.reshape(T * K)

    oh = (e_flat[:, None] == jnp.arange(E, dtype=jnp.int32)).astype(jnp.int32)
    ranks = jnp.cumsum(oh, axis=0)
    counts = ranks[-1]
    rank = jnp.take_along_axis(ranks, e_flat[:, None], axis=1)[:, 0] - 1
    tiles_per_e = (counts + M - 1) // M
    seg_starts = jnp.concatenate(
        [jnp.zeros(1, jnp.int32),
         jnp.cumsum(tiles_per_e).astype(jnp.int32)]) * M
    pos = seg_starts[e_flat] + rank
    arange_a = jnp.arange(T * K, dtype=jnp.int32)
    row_tok = jnp.zeros(NPAD, jnp.int32).at[pos].set(arange_a // K)
    ws = jnp.zeros(NPAD, jnp.float32).at[pos].set(wf)
    num_real = jnp.sum(tiles_per_e).astype(jnp.int32)
    tile_ids = jnp.arange(NT, dtype=jnp.int32)
    tile_eid = jnp.minimum(
        jnp.searchsorted(seg_starts[1:], tile_ids * M, side="right"),
        E - 1).astype(jnp.int32)
    tile_valid = (tile_ids < num_real).astype(jnp.int32)
    meta = jnp.concatenate([tile_eid, tile_valid])

    xs = _sc_dispatch(x_flat, row_tok)

    y = _grouped_ffn(meta, xs, W1, b1, W2, b2, ws.reshape(NPAD, 1))

    pos_tk = pos.reshape(T, K)
    out = _sc_combine(y, pos_tk[:, 0], pos_tk[:, 1])
    return out.reshape(B, S, D)


# speedup vs baseline: 1.2758x; 1.2196x over previous
import functools

import jax
import jax.numpy as jnp
from jax import lax
from jax.experimental import pallas as pl
from jax.experimental.pallas import tpu as pltpu
from jax.experimental.pallas import tpu_sc as plsc

E = 8
K = 2
B = 2
S = 2048
D = 1024
F = 4096
T = B * S

M = 256
FB = 2048
NF = F // FB
NT = 40
NPAD = NT * M


def _gate_body(x_ref, wg_ref, bg_ref, w_ref, idx_ref):
    x = x_ref[0]
    logits = jax.lax.dot_general(
        x, wg_ref[...], (((1,), (0,)), ((), ())),
        preferred_element_type=jnp.float32) + bg_ref[...]
    iota = jax.lax.broadcasted_iota(jnp.int32, logits.shape, 1)
    m1 = jnp.max(logits, axis=1, keepdims=True)
    a1 = jnp.min(jnp.where(logits == m1, iota, E), axis=1, keepdims=True)
    l2 = jnp.where(iota == a1, -jnp.inf, logits)
    m2 = jnp.max(l2, axis=1, keepdims=True)
    a2 = jnp.min(jnp.where(l2 == m2, iota, E), axis=1, keepdims=True)
    e1 = jnp.exp(m1 - jnp.max(m1, axis=0, keepdims=True))
    e2 = jnp.exp(m2 - jnp.max(m2, axis=0, keepdims=True))
    w_ref[0, :, 0:1] = e1 / jnp.sum(e1, axis=0, keepdims=True)
    w_ref[0, :, 1:2] = e2 / jnp.sum(e2, axis=0, keepdims=True)
    idx_ref[0, :, 0:1] = a1
    idx_ref[0, :, 1:2] = a2


def _route(inputs, Wg, bg):
    return pl.pallas_call(
        _gate_body,
        grid=(B,),
        in_specs=[
            pl.BlockSpec((1, S, D), lambda b: (b, 0, 0)),
            pl.BlockSpec((D, E), lambda b: (0, 0)),
            pl.BlockSpec((1, E), lambda b: (0, 0)),
        ],
        out_specs=[
            pl.BlockSpec((1, S, K), lambda b: (b, 0, 0)),
            pl.BlockSpec((1, S, K), lambda b: (b, 0, 0)),
        ],
        out_shape=[
            jax.ShapeDtypeStruct((B, S, K), jnp.float32),
            jax.ShapeDtypeStruct((B, S, K), jnp.int32),
        ],
    )(inputs, Wg, bg.reshape(1, E))


def _ffn_body(meta_ref, xs_ref, w1_ref, b1_ref, w2_ref, b2_ref, y_ref):
    t = pl.program_id(0)
    f = pl.program_id(1)

    @pl.when(meta_ref[NT + t] == 1)
    def _():
        x = xs_ref[...].astype(jnp.bfloat16)
        h = jax.lax.dot_general(
            x, w1_ref[0].astype(jnp.bfloat16), (((1,), (0,)), ((), ())),
            preferred_element_type=jnp.float32) + b1_ref[0]
        h = 0.5 * h * (1.0 + jax.lax.erf(h * 0.7071067811865476))
        part = jax.lax.dot_general(
            h.astype(jnp.bfloat16), w2_ref[0].astype(jnp.bfloat16),
            (((1,), (0,)), ((), ())),
            preferred_element_type=jnp.float32)

        @pl.when(f == 0)
        def _():
            y_ref[...] = part + b2_ref[0]

        @pl.when(f > 0)
        def _():
            y_ref[...] = y_ref[...] + part


def _grouped_ffn(meta, xs, W1, b1, W2, b2):
    grid_spec = pltpu.PrefetchScalarGridSpec(
        num_scalar_prefetch=1,
        grid=(NT, NF),
        in_specs=[
            pl.BlockSpec((M, D), lambda t, f, m: (t, 0)),
            pl.BlockSpec((1, D, FB), lambda t, f, m: (m[t], 0, f)),
            pl.BlockSpec((1, 1, FB), lambda t, f, m: (m[t], 0, f)),
            pl.BlockSpec((1, FB, D), lambda t, f, m: (m[t], f, 0)),
            pl.BlockSpec((1, 1, D), lambda t, f, m: (m[t], 0, 0)),
        ],
        out_specs=pl.BlockSpec((M, D), lambda t, f, m: (t, 0)),
    )
    return pl.pallas_call(
        _ffn_body,
        grid_spec=grid_spec,
        out_shape=jax.ShapeDtypeStruct((NPAD, D), jnp.float32),
        compiler_params=pltpu.CompilerParams(
            dimension_semantics=("arbitrary", "arbitrary")),
    )(meta, xs, W1, b1.reshape(E, 1, F), W2, b2.reshape(E, 1, D))


NW = 32
TPW = T // NW
DCH = 64
CCH = 32

_sc_mesh = plsc.VectorSubcoreMesh(core_axis_name="c", subcore_axis_name="s")


@functools.partial(
    pl.kernel, mesh=_sc_mesh,
    out_type=jax.ShapeDtypeStruct((NPAD, D), jnp.float32),
    scratch_types=[
        pltpu.VMEM((DCH,), jnp.int32),
        pltpu.VMEM((DCH,), jnp.int32),
        pltpu.VMEM((DCH, D), jnp.float32),
        pltpu.SemaphoreType.DMA,
        pltpu.SemaphoreType.DMA,
    ],
)
def _sc_dispatch(x_hbm, p0_hbm, p1_hbm, xs_hbm, i0_v, i1_v, rows_v, s0, s1):
    wid = lax.axis_index("s") * 2 + lax.axis_index("c")
    base = wid * TPW
    for c in range(TPW // DCH):
        off = base + c * DCH
        pltpu.sync_copy(p0_hbm.at[pl.ds(off, DCH)], i0_v)
        pltpu.sync_copy(p1_hbm.at[pl.ds(off, DCH)], i1_v)
        pltpu.sync_copy(x_hbm.at[pl.ds(off, DCH)], rows_v)
        cp0 = pltpu.async_copy(rows_v, xs_hbm.at[i0_v], s0)
        cp1 = pltpu.async_copy(rows_v, xs_hbm.at[i1_v], s1)
        cp0.wait()
        cp1.wait()


@functools.partial(
    pl.kernel, mesh=_sc_mesh,
    out_type=jax.ShapeDtypeStruct((T, D), jnp.float32),
    scratch_types=[
        pltpu.VMEM((CCH,), jnp.int32),
        pltpu.VMEM((CCH,), jnp.int32),
        pltpu.VMEM((CCH, 16), jnp.float32),
        pltpu.VMEM((CCH, 16), jnp.float32),
        pltpu.VMEM((CCH, D), jnp.float32),
        pltpu.VMEM((CCH, D), jnp.float32),
        pltpu.SemaphoreType.DMA,
        pltpu.SemaphoreType.DMA,
    ],
)
def _sc_combine(y_hbm, p0_hbm, p1_hbm, w0_hbm, w1_hbm, out_hbm,
                i0_v, i1_v, w0_v, w1_v, y0_v, y1_v, s0, s1):
    wid = lax.axis_index("s") * 2 + lax.axis_index("c")
    base = wid * TPW
    for c in range(TPW // CCH):
        off = base + c * CCH
        pltpu.sync_copy(p0_hbm.at[pl.ds(off, CCH)], i0_v)
        pltpu.sync_copy(p1_hbm.at[pl.ds(off, CCH)], i1_v)
        pltpu.sync_copy(w0_hbm.at[pl.ds(off, CCH)], w0_v)
        pltpu.sync_copy(w1_hbm.at[pl.ds(off, CCH)], w1_v)
        cp0 = pltpu.async_copy(y_hbm.at[i0_v], y0_v, s0)
        cp1 = pltpu.async_copy(y_hbm.at[i1_v], y1_v, s1)
        cp0.wait()
        cp1.wait()

        def _row(r, carry):
            w0 = w0_v[r]
            w1 = w1_v[r]
            for j in range(D // 16):
                sl = pl.ds(j * 16, 16)
                y0_v[r, sl] = y0_v[r, sl] * w0 + y1_v[r, sl] * w1
            return carry

        lax.fori_loop(0, CCH, _row, 0)
        pltpu.sync_copy(y0_v, out_hbm.at[pl.ds(off, CCH)])


def kernel(inputs, Wg, bg, W1, b1, W2, b2):
    x_flat = inputs.reshape(T, D)

    w_bsk, idx_bsk = _route(inputs, Wg, bg)
    wf = w_bsk.reshape(T * K)
    e_flat = idx_bsk.reshape(T * K)

    oh = (e_flat[:, None] == jnp.arange(E, dtype=jnp.int32)).astype(jnp.int32)
    ranks = jnp.cumsum(oh, axis=0)
    counts = ranks[-1]
    rank = jnp.take_along_axis(ranks, e_flat[:, None], axis=1)[:, 0] - 1
    tiles_per_e = (counts + M - 1) // M
    seg_starts = jnp.concatenate(
        [jnp.zeros(1, jnp.int32),
         jnp.cumsum(tiles_per_e).astype(jnp.int32)]) * M
    pos = seg_starts[e_flat] + rank
    num_real = jnp.sum(tiles_per_e).astype(jnp.int32)
    tile_ids = jnp.arange(NT, dtype=jnp.int32)
    tile_eid = jnp.minimum(
        jnp.searchsorted(seg_starts[1:], tile_ids * M, side="right"),
        E - 1).astype(jnp.int32)
    tile_valid = (tile_ids < num_real).astype(jnp.int32)
    meta = jnp.concatenate([tile_eid, tile_valid])

    pos_tk = pos.reshape(T, K)
    p0 = pos_tk[:, 0]
    p1 = pos_tk[:, 1]
    wf_tk = wf.reshape(T, K)
    w0_b = jnp.broadcast_to(wf_tk[:, 0:1], (T, 16))
    w1_b = jnp.broadcast_to(wf_tk[:, 1:2], (T, 16))

    xs = _sc_dispatch(x_flat, p0, p1)

    y = _grouped_ffn(meta, xs, W1, b1, W2, b2)

    out = _sc_combine(y, p0, p1, w0_b, w1_b)
    return out.reshape(B, S, D)


# speedup vs baseline: 1.3439x; 1.0533x over previous
import functools

import jax
import jax.numpy as jnp
from jax import lax
from jax.experimental import pallas as pl
from jax.experimental.pallas import tpu as pltpu
from jax.experimental.pallas import tpu_sc as plsc

E = 8
K = 2
B = 2
S = 2048
D = 1024
F = 4096
T = B * S

M = 256
FB = 2048
NF = F // FB
NT = 40
NPAD = NT * M


def _gate_body(x_ref, wg_ref, bg_ref, w_ref, idx_ref):
    x = x_ref[0]
    logits = jax.lax.dot_general(
        x, wg_ref[...], (((1,), (0,)), ((), ())),
        preferred_element_type=jnp.float32) + bg_ref[...]
    iota = jax.lax.broadcasted_iota(jnp.int32, logits.shape, 1)
    m1 = jnp.max(logits, axis=1, keepdims=True)
    a1 = jnp.min(jnp.where(logits == m1, iota, E), axis=1, keepdims=True)
    l2 = jnp.where(iota == a1, -jnp.inf, logits)
    m2 = jnp.max(l2, axis=1, keepdims=True)
    a2 = jnp.min(jnp.where(l2 == m2, iota, E), axis=1, keepdims=True)
    e1 = jnp.exp(m1 - jnp.max(m1, axis=0, keepdims=True))
    e2 = jnp.exp(m2 - jnp.max(m2, axis=0, keepdims=True))
    w_ref[0, :, 0:1] = e1 / jnp.sum(e1, axis=0, keepdims=True)
    w_ref[0, :, 1:2] = e2 / jnp.sum(e2, axis=0, keepdims=True)
    idx_ref[0, :, 0:1] = a1
    idx_ref[0, :, 1:2] = a2


def _route(inputs, Wg, bg):
    return pl.pallas_call(
        _gate_body,
        grid=(B,),
        in_specs=[
            pl.BlockSpec((1, S, D), lambda b: (b, 0, 0)),
            pl.BlockSpec((D, E), lambda b: (0, 0)),
            pl.BlockSpec((1, E), lambda b: (0, 0)),
        ],
        out_specs=[
            pl.BlockSpec((1, S, K), lambda b: (b, 0, 0)),
            pl.BlockSpec((1, S, K), lambda b: (b, 0, 0)),
        ],
        out_shape=[
            jax.ShapeDtypeStruct((B, S, K), jnp.float32),
            jax.ShapeDtypeStruct((B, S, K), jnp.int32),
        ],
    )(inputs, Wg, bg.reshape(1, E))


def _ffn_body(meta_ref, xs_ref, w1_ref, b1_ref, w2_ref, b2_ref, y_ref):
    f = pl.program_id(0)
    t = pl.program_id(1)

    @pl.when(meta_ref[NT + t] == 1)
    def _():
        x = xs_ref[...].astype(jnp.bfloat16)
        h = jax.lax.dot_general(
            x, w1_ref[0].astype(jnp.bfloat16), (((1,), (0,)), ((), ())),
            preferred_element_type=jnp.float32) + b1_ref[0]
        h = 0.5 * h * (1.0 + jax.lax.erf(h * 0.7071067811865476))
        part = jax.lax.dot_general(
            h.astype(jnp.bfloat16), w2_ref[0].astype(jnp.bfloat16),
            (((1,), (0,)), ((), ())),
            preferred_element_type=jnp.float32)

        @pl.when(f == 0)
        def _():
            y_ref[0] = part + b2_ref[0]

        @pl.when(f > 0)
        def _():
            y_ref[0] = part


def _grouped_ffn(meta, xs, W1, b1, W2, b2):
    grid_spec = pltpu.PrefetchScalarGridSpec(
        num_scalar_prefetch=1,
        grid=(NF, NT),
        in_specs=[
            pl.BlockSpec((M, D), lambda f, t, m: (t, 0)),
            pl.BlockSpec((1, D, FB), lambda f, t, m: (m[t], 0, f)),
            pl.BlockSpec((1, 1, FB), lambda f, t, m: (m[t], 0, f)),
            pl.BlockSpec((1, FB, D), lambda f, t, m: (m[t], f, 0)),
            pl.BlockSpec((1, 1, D), lambda f, t, m: (m[t], 0, 0)),
        ],
        out_specs=pl.BlockSpec((1, M, D), lambda f, t, m: (f, t, 0)),
    )
    return pl.pallas_call(
        _ffn_body,
        grid_spec=grid_spec,
        out_shape=jax.ShapeDtypeStruct((NF, NPAD, D), jnp.float32),
        compiler_params=pltpu.CompilerParams(
            dimension_semantics=("arbitrary", "arbitrary")),
    )(meta, xs, W1, b1.reshape(E, 1, F), W2, b2.reshape(E, 1, D))


NW = 32
TPW = T // NW
DCH = 64
CCH = 16

_sc_mesh = plsc.VectorSubcoreMesh(core_axis_name="c", subcore_axis_name="s")


@functools.partial(
    pl.kernel, mesh=_sc_mesh,
    out_type=jax.ShapeDtypeStruct((NPAD, D), jnp.float32),
    scratch_types=[
        pltpu.VMEM((DCH,), jnp.int32),
        pltpu.VMEM((DCH,), jnp.int32),
        pltpu.VMEM((DCH, D), jnp.float32),
        pltpu.SemaphoreType.DMA,
        pltpu.SemaphoreType.DMA,
    ],
)
def _sc_dispatch(x_hbm, p0_hbm, p1_hbm, xs_hbm, i0_v, i1_v, rows_v, s0, s1):
    wid = lax.axis_index("s") * 2 + lax.axis_index("c")
    base = wid * TPW
    for c in range(TPW // DCH):
        off = base + c * DCH
        pltpu.sync_copy(p0_hbm.at[pl.ds(off, DCH)], i0_v)
        pltpu.sync_copy(p1_hbm.at[pl.ds(off, DCH)], i1_v)
        pltpu.sync_copy(x_hbm.at[pl.ds(off, DCH)], rows_v)
        cp0 = pltpu.async_copy(rows_v, xs_hbm.at[i0_v], s0)
        cp1 = pltpu.async_copy(rows_v, xs_hbm.at[i1_v], s1)
        cp0.wait()
        cp1.wait()


@functools.partial(
    pl.kernel, mesh=_sc_mesh,
    out_type=jax.ShapeDtypeStruct((T, D), jnp.float32),
    scratch_types=[
        pltpu.VMEM((CCH,), jnp.int32),
        pltpu.VMEM((CCH,), jnp.int32),
        pltpu.VMEM((CCH, 16), jnp.float32),
        pltpu.VMEM((CCH, 16), jnp.float32),
        pltpu.VMEM((CCH, D), jnp.float32),
        pltpu.VMEM((CCH, D), jnp.float32),
        pltpu.VMEM((CCH, D), jnp.float32),
        pltpu.VMEM((CCH, D), jnp.float32),
        pltpu.SemaphoreType.DMA,
        pltpu.SemaphoreType.DMA,
        pltpu.SemaphoreType.DMA,
        pltpu.SemaphoreType.DMA,
    ],
)
def _sc_combine(ya_hbm, yb_hbm, p0_hbm, p1_hbm, w0_hbm, w1_hbm, out_hbm,
                i0_v, i1_v, w0_v, w1_v, a0_v, b0_v, a1_v, b1_v,
                s0, s1, s2, s3):
    wid = lax.axis_index("s") * 2 + lax.axis_index("c")
    base = wid * TPW
    for c in range(TPW // CCH):
        off = base + c * CCH
        pltpu.sync_copy(p0_hbm.at[pl.ds(off, CCH)], i0_v)
        pltpu.sync_copy(p1_hbm.at[pl.ds(off, CCH)], i1_v)
        pltpu.sync_copy(w0_hbm.at[pl.ds(off, CCH)], w0_v)
        pltpu.sync_copy(w1_hbm.at[pl.ds(off, CCH)], w1_v)
        cp0 = pltpu.async_copy(ya_hbm.at[i0_v], a0_v, s0)
        cp1 = pltpu.async_copy(yb_hbm.at[i0_v], b0_v, s1)
        cp2 = pltpu.async_copy(ya_hbm.at[i1_v], a1_v, s2)
        cp3 = pltpu.async_copy(yb_hbm.at[i1_v], b1_v, s3)
        cp0.wait()
        cp1.wait()
        cp2.wait()
        cp3.wait()

        def _row(r, carry):
            w0 = w0_v[r]
            w1 = w1_v[r]
            for j in range(D // 16):
                sl = pl.ds(j * 16, 16)
                a0_v[r, sl] = (a0_v[r, sl] + b0_v[r, sl]) * w0 + \
                    (a1_v[r, sl] + b1_v[r, sl]) * w1
            return carry

        lax.fori_loop(0, CCH, _row, 0)
        pltpu.sync_copy(a0_v, out_hbm.at[pl.ds(off, CCH)])


def kernel(inputs, Wg, bg, W1, b1, W2, b2):
    x_flat = inputs.reshape(T, D)

    w_bsk, idx_bsk = _route(inputs, Wg, bg)
    wf = w_bsk.reshape(T * K)
    e_flat = idx_bsk.reshape(T * K)

    oh = (e_flat[:, None] == jnp.arange(E, dtype=jnp.int32)).astype(jnp.int32)
    ranks = jnp.cumsum(oh, axis=0)
    counts = ranks[-1]
    rank = jnp.take_along_axis(ranks, e_flat[:, None], axis=1)[:, 0] - 1
    tiles_per_e = (counts + M - 1) // M
    seg_starts = jnp.concatenate(
        [jnp.zeros(1, jnp.int32),
         jnp.cumsum(tiles_per_e).astype(jnp.int32)]) * M
    pos = seg_starts[e_flat] + rank
    num_real = jnp.sum(tiles_per_e).astype(jnp.int32)
    tile_ids = jnp.arange(NT, dtype=jnp.int32)
    tile_eid = jnp.minimum(
        jnp.searchsorted(seg_starts[1:], tile_ids * M, side="right"),
        E - 1).astype(jnp.int32)
    tile_valid = (tile_ids < num_real).astype(jnp.int32)
    meta = jnp.concatenate([tile_eid, tile_valid])

    pos_tk = pos.reshape(T, K)
    p0 = pos_tk[:, 0]
    p1 = pos_tk[:, 1]
    wf_tk = wf.reshape(T, K)
    w0_b = jnp.broadcast_to(wf_tk[:, 0:1], (T, 16))
    w1_b = jnp.broadcast_to(wf_tk[:, 1:2], (T, 16))

    xs = _sc_dispatch(x_flat, p0, p1)

    y = _grouped_ffn(meta, xs, W1, b1, W2, b2)

    out = _sc_combine(y[0], y[1], p0, p1, w0_b, w1_b)
    return out.reshape(B, S, D)
